# baseline (speedup 1.0000x reference)
import functools

import jax
import jax.numpy as jnp
from jax import lax
from jax.experimental import pallas as pl
from jax.experimental.pallas import tpu as pltpu
from jax.experimental.pallas import tpu_sc as plsc

_VOCAB = 49408
_D = 512
_NC = 1000
_CTX_LEN = 77
_N_CTX = 16
_SUF = _CTX_LEN - (_N_CTX + 1)
_SUF_PAD = 64
_PREFIX_ROWS = _CTX_LEN - _SUF

_NW = 32
_CPW = 32
_NC_PAD = _NW * _CPW

_JD = _D // 128
_NU_SUF = _SUF * _JD
_NSPLIT = 3
_USPL = _NU_SUF // _NSPLIT
_ROW_TILES = _NC // 8
_UNITS_PER_T = _ROW_TILES * _JD * 8
_PFX_WORKERS = 25
_PFX_CHUNK = _UNITS_PER_T // _PFX_WORKERS
_TMPL_REPS = 5
_TMPL_U = 32 * _TMPL_REPS
_CTX_GROUPS = _ROW_TILES // _TMPL_REPS


def _emb_body(tab_u, prompts_f, pfx_u, ctx_u, out_u, eos,
              gbuf0, gbuf1, gidx0, gidx1, sidx0, sidx1, sbase,
              idx_all, tmpl, cidx, pfx_v, eos_v,
              sem_ids, sem_g0, sem_g1, sem_s0, sem_s1,
              sem_ctx, sem_pfx, sem_eos):
    wid = lax.axis_index("s") * 2 + lax.axis_index("c")
    base = wid * _CPW
    n_cls = jnp.minimum(jnp.int32(_CPW), jnp.int32(_NC) - base)
    gbufs = (gbuf0, gbuf1)
    gidxs = (gidx0, gidx1)
    sidxs = (sidx0, sidx1)
    sem_g = (sem_g0, sem_g1)
    sem_s = (sem_s0, sem_s1)

    lanes = lax.iota(jnp.int32, 16)

    ids_off = pl.multiple_of(base * _SUF_PAD, 8)
    pltpu.async_copy(prompts_f.at[pl.ds(ids_off, _CPW * _SUF_PAD)], idx_all,
                     sem_ids)

    pfx_off = jnp.minimum(wid, jnp.int32(_PFX_WORKERS - 1)) * _PFX_CHUNK

    @pl.when(wid < _PFX_WORKERS)
    def _():
        pltpu.async_copy(pfx_u.at[pl.ds(pfx_off, _PFX_CHUNK)], pfx_v, sem_pfx)

    tct = wid >> 1
    half = wid & 1
    for j in range(2):
        for m in range(_TMPL_U // 32):
            r = 80 * j + 16 * m + lanes
            cidx[j, pl.ds(16 * m, 16)] = (
                32 * (tct >> 3) + 8 * ((r & 31) >> 3) + (tct & 7))
    for j in range(2):
        pltpu.async_copy(ctx_u.at[cidx.at[j]],
                         tmpl.at[pl.ds(80 * j, 80)], sem_ctx)
    for j in range(2):
        pltpu.make_async_copy(ctx_u.at[cidx.at[j]],
                              tmpl.at[pl.ds(80 * j, 80)], sem_ctx).wait()

    n_grp = jnp.int32(13 - half)
    grp0 = jnp.int32(13) * half
    t_base = jnp.int32(_UNITS_PER_T) * (tct + 1) + _TMPL_U * grp0

    def ctx_rep(g, carry):
        off = pl.multiple_of(t_base + _TMPL_U * g, 8)
        pltpu.async_copy(tmpl, out_u.at[pl.ds(off, _TMPL_U)], sem_ctx)
        return carry

    lax.fori_loop(0, n_grp, ctx_rep, 0)

    for j in range(_NSPLIT):
        for m in range(_USPL // 16):
            k = 80 * j + 16 * m + lanes
            sbase[j, pl.ds(16 * m, 16)] = (
                jnp.int32(_UNITS_PER_T) * (_PREFIX_ROWS + (k >> 2))
                + 8 * (k & 3))

    @pl.when(wid < _PFX_WORKERS)
    def _():
        pltpu.make_async_copy(pfx_u.at[pl.ds(pfx_off, _PFX_CHUNK)], pfx_v,
                              sem_pfx).wait()
        pltpu.async_copy(pfx_v, out_u.at[pl.ds(pfx_off, _PFX_CHUNK)], sem_pfx)

    pltpu.make_async_copy(prompts_f.at[pl.ds(ids_off, _CPW * _SUF_PAD)],
                          idx_all, sem_ids).wait()

    def one_class(i, b):
        c = jnp.minimum(base + i, jnp.int32(_NC - 1))
        i_loc = jnp.minimum(i, n_cls - 1)
        row0 = i_loc * _SUF_PAD
        offc = 32 * (c >> 3) + (c & 7)
        @pl.when(i >= 2)
        def _():
            for j in range(_NSPLIT):
                pltpu.make_async_copy(gbufs[b].at[j], out_u.at[sidxs[b].at[j]],
                                      sem_s[b]).wait()
        for j in range(_NSPLIT):
            for m in range(_USPL // 16):
                k = 80 * j + 16 * m + lanes
                x = plsc.load_gather(idx_all, [row0 + (k >> 2)])
                gidxs[b][j, pl.ds(16 * m, 16)] = (
                    32 * (x >> 3) + 8 * (k & 3) + (x & 7))
                sidxs[b][j, pl.ds(16 * m, 16)] = (
                    sbase[j, pl.ds(16 * m, 16)] + offc)
            pltpu.async_copy(tab_u.at[gidxs[b].at[j]], gbufs[b].at[j],
                             sem_g[b])

        best_val = jnp.int32(-2)
        best_pos = jnp.int32(0)
        for m in range(_SUF_PAD // 16):
            voff = pl.multiple_of(row0 + 16 * m, 8)
            vj = idx_all[pl.ds(voff, 16)]
            mj = jnp.max(vj)
            pj = jnp.min(jnp.where(vj == mj, lanes + jnp.int32(16 * m),
                                   jnp.int32(1 << 20)))
            upd = mj > best_val
            best_pos = jnp.where(upd, pj, best_pos)
            best_val = jnp.where(upd, mj, best_val)
        plsc.store_scatter(
            eos_v,
            [jnp.full((16,), i, jnp.int32)],
            jnp.full((16,), best_pos + jnp.int32(_PREFIX_ROWS), jnp.int32),
            mask=lanes == 0,
        )

        for j in range(_NSPLIT):
            pltpu.make_async_copy(tab_u.at[gidxs[b].at[j]], gbufs[b].at[j],
                                  sem_g[b]).wait()
        for j in range(_NSPLIT):
            pltpu.async_copy(gbufs[b].at[j], out_u.at[sidxs[b].at[j]],
                             sem_s[b])

    def pair(k, carry):
        one_class(2 * k, 0)
        one_class(2 * k + 1, 1)
        return carry

    lax.fori_loop(0, _CPW // 2, pair, 0)

    for b in range(2):
        for j in range(_NSPLIT):
            pltpu.make_async_copy(gbufs[b].at[j], out_u.at[sidxs[b].at[j]],
                                  sem_s[b]).wait()

    def ctx_drain(g, carry):
        pltpu.make_async_copy(tmpl, out_u.at[pl.ds(0, _TMPL_U)],
                              sem_ctx).wait()
        return carry

    lax.fori_loop(0, n_grp, ctx_drain, 0)

    @pl.when(wid < _PFX_WORKERS)
    def _():
        pltpu.make_async_copy(pfx_v, out_u.at[pl.ds(pfx_off, _PFX_CHUNK)],
                              sem_pfx).wait()

    pltpu.async_copy(eos_v, eos.at[pl.ds(base, _CPW)], sem_eos).wait()


@functools.partial(jax.jit, static_argnames=())
def _emb_call(tab_u, prompts_f, pfx_u, ctx_u):
    mesh = plsc.VectorSubcoreMesh(core_axis_name="c", subcore_axis_name="s")
    return pl.kernel(
        _emb_body,
        out_type=[
            jax.ShapeDtypeStruct((_CTX_LEN * _UNITS_PER_T, 128), jnp.float32),
            jax.ShapeDtypeStruct((_NC_PAD,), jnp.int32),
        ],
        mesh=mesh,
        scratch_types=[
            pltpu.VMEM((_NSPLIT, _USPL, 128), jnp.float32),
            pltpu.VMEM((_NSPLIT, _USPL, 128), jnp.float32),
            pltpu.VMEM((_NSPLIT, _USPL), jnp.int32),
            pltpu.VMEM((_NSPLIT, _USPL), jnp.int32),
            pltpu.VMEM((_NSPLIT, _USPL), jnp.int32),
            pltpu.VMEM((_NSPLIT, _USPL), jnp.int32),
            pltpu.VMEM((_NSPLIT, _USPL), jnp.int32),
            pltpu.VMEM((_CPW * _SUF_PAD,), jnp.int32),
            pltpu.VMEM((_TMPL_U, 128), jnp.float32),
            pltpu.VMEM((2, 80), jnp.int32),
            pltpu.VMEM((_PFX_CHUNK, 128), jnp.float32),
            pltpu.VMEM((_CPW,), jnp.int32),
            pltpu.SemaphoreType.DMA,
            pltpu.SemaphoreType.DMA,
            pltpu.SemaphoreType.DMA,
            pltpu.SemaphoreType.DMA,
            pltpu.SemaphoreType.DMA,
            pltpu.SemaphoreType.DMA,
            pltpu.SemaphoreType.DMA,
            pltpu.SemaphoreType.DMA,
        ],
        compiler_params=pltpu.CompilerParams(use_tc_tiling_on_sc=False,
                                             needs_layout_passes=False),
    )(tab_u, prompts_f, pfx_u, ctx_u)


def kernel(token_embedding, prompts, token_prefix, ctx_embedding):
    tab_u = (token_embedding.reshape(_VOCAB // 8, 8, _JD, 128)
             .transpose(0, 2, 1, 3).reshape(_VOCAB * _JD, 128))
    pfx_u = (token_prefix.reshape(_NC // 8, 8, _JD, 128)
             .transpose(0, 2, 1, 3).reshape(_NC * _JD, 128))
    ctx_u = (ctx_embedding.reshape(_N_CTX // 8, 8, _JD, 128)
             .transpose(0, 2, 1, 3).reshape(_N_CTX * _JD, 128))
    prompts_i = prompts.astype(jnp.int32)
    prompts_f = jnp.pad(prompts_i, ((0, 0), (0, _SUF_PAD - _SUF)),
                        constant_values=-1).reshape(_NC * _SUF_PAD)
    prompts_f = jnp.pad(prompts_f, (0, (_NC_PAD - _NC) * _SUF_PAD),
                        constant_values=-1)
    out_u, eos = _emb_call(tab_u, prompts_f, pfx_u, ctx_u)
    emb = (out_u.reshape(_CTX_LEN, _ROW_TILES, _JD, 8, 128)
           .transpose(1, 3, 0, 2, 4).reshape(_NC, _CTX_LEN, _D))
    return emb, eos[:_NC]


# speedup vs baseline: 1.0635x; 1.0635x over previous
import functools

import jax
import jax.numpy as jnp
from jax import lax
from jax.experimental import pallas as pl
from jax.experimental.pallas import tpu as pltpu
from jax.experimental.pallas import tpu_sc as plsc

_VOCAB = 49408
_D = 512
_NC = 1000
_CTX_LEN = 77
_N_CTX = 16
_SUF = _CTX_LEN - (_N_CTX + 1)
_SUF_PAD = 64
_PREFIX_ROWS = _CTX_LEN - _SUF

_NW = 32
_CPW = 32
_NC_PAD = _NW * _CPW

_JD = _D // 128
_NU_SUF = _SUF * _JD
_NSPLIT = 3
_USPL = _NU_SUF // _NSPLIT
_ROW_TILES = _NC // 8
_UNITS_PER_T = _ROW_TILES * _JD * 8
_PFX_WORKERS = 25
_PFX_CHUNK = _UNITS_PER_T // _PFX_WORKERS


def _emb_body(tab_u, prompts_p, pfx_u, ctx_u, out_u, eos,
              gbuf0, gbuf1, gbuf2, gidx0, gidx1, gidx2,
              sidx0, sidx1, sidx2, sbase,
              idx0, idx1, idx2, tmpl, cidx, pfx_v, eos_v,
              sem_i0, sem_i1, sem_i2, sem_g0, sem_g1, sem_g2,
              sem_s0, sem_s1, sem_s2,
              sem_ctx, sem_pfx, sem_eos):
    wid = lax.axis_index("s") * 2 + lax.axis_index("c")
    base = wid * _CPW
    gbufs = (gbuf0, gbuf1, gbuf2)
    gidxs = (gidx0, gidx1, gidx2)
    sidxs = (sidx0, sidx1, sidx2)
    idxs = (idx0, idx1, idx2)
    sem_i = (sem_i0, sem_i1, sem_i2)
    sem_g = (sem_g0, sem_g1, sem_g2)
    sem_s = (sem_s0, sem_s1, sem_s2)

    lanes = lax.iota(jnp.int32, 16)

    def cls(i):
        return jnp.minimum(base + i, jnp.int32(_NC - 1))

    pfx_off = jnp.minimum(wid, jnp.int32(_PFX_WORKERS - 1)) * _PFX_CHUNK

    @pl.when(wid < _PFX_WORKERS)
    def _():
        pltpu.async_copy(pfx_u.at[pl.ds(pfx_off, _PFX_CHUNK)], pfx_v, sem_pfx)

    tct = wid >> 1
    half = wid & 1
    for m in range(2):
        r = lanes + 16 * m
        u = 32 * (tct >> 3) + 8 * (r >> 3) + (tct & 7)
        cidx[pl.ds(16 * m, 16)] = u
    pltpu.async_copy(ctx_u.at[cidx], tmpl, sem_ctx).wait()

    n_rep = jnp.int32(63 - half)
    rep0 = jnp.int32(63) * half
    t_base = jnp.int32(_UNITS_PER_T) * (tct + 1) + 32 * rep0

    def ctx_rep(r, carry):
        off = pl.multiple_of(t_base + 32 * r, 8)
        pltpu.async_copy(tmpl, out_u.at[pl.ds(off, 32)], sem_ctx)
        return carry

    lax.fori_loop(0, n_rep, ctx_rep, 0)

    for j in range(_NSPLIT):
        for m in range(_USPL // 16):
            k = 80 * j + 16 * m + lanes
            sbase[j, pl.ds(16 * m, 16)] = (
                jnp.int32(_UNITS_PER_T) * (_PREFIX_ROWS + (k >> 2))
                + 8 * (k & 3))

    @pl.when(wid < _PFX_WORKERS)
    def _():
        pltpu.make_async_copy(pfx_u.at[pl.ds(pfx_off, _PFX_CHUNK)], pfx_v,
                              sem_pfx).wait()
        pltpu.async_copy(pfx_v, out_u.at[pl.ds(pfx_off, _PFX_CHUNK)], sem_pfx)

    pltpu.async_copy(prompts_p.at[cls(0)], idx0, sem_i0)

    def one_class(i, b):
        c = cls(i)
        offc = 32 * (c >> 3) + (c & 7)
        @pl.when(i >= 3)
        def _():
            for j in range(_NSPLIT):
                pltpu.make_async_copy(gbufs[b].at[j], out_u.at[sidxs[b].at[j]],
                                      sem_s[b]).wait()
        pltpu.make_async_copy(prompts_p.at[c], idxs[b], sem_i[b]).wait()
        for j in range(_NSPLIT):
            for m in range(_USPL // 16):
                k = 80 * j + 16 * m + lanes
                x = plsc.load_gather(idxs[b], [k >> 2])
                gidxs[b][j, pl.ds(16 * m, 16)] = (
                    32 * (x >> 3) + 8 * (k & 3) + (x & 7))
                sidxs[b][j, pl.ds(16 * m, 16)] = (
                    sbase[j, pl.ds(16 * m, 16)] + offc)
            pltpu.async_copy(tab_u.at[gidxs[b].at[j]], gbufs[b].at[j],
                             sem_g[b])
        nb = (b + 1) % 3
        pltpu.async_copy(prompts_p.at[cls(i + 1)], idxs[nb], sem_i[nb])

        best_val = jnp.int32(-2)
        best_pos = jnp.int32(0)
        for m in range(_SUF_PAD // 16):
            vj = idxs[b][pl.ds(16 * m, 16)]
            mj = jnp.max(vj)
            pj = jnp.min(jnp.where(vj == mj, lanes + jnp.int32(16 * m),
                                   jnp.int32(1 << 20)))
            upd = mj > best_val
            best_pos = jnp.where(upd, pj, best_pos)
            best_val = jnp.where(upd, mj, best_val)
        plsc.store_scatter(
            eos_v,
            [jnp.full((16,), i, jnp.int32)],
            jnp.full((16,), best_pos + jnp.int32(_PREFIX_ROWS), jnp.int32),
            mask=lanes == 0,
        )

        for j in range(_NSPLIT):
            pltpu.make_async_copy(tab_u.at[gidxs[b].at[j]], gbufs[b].at[j],
                                  sem_g[b]).wait()
        for j in range(_NSPLIT):
            pltpu.async_copy(gbufs[b].at[j], out_u.at[sidxs[b].at[j]],
                             sem_s[b])

    def triple(k, carry):
        one_class(3 * k, 0)
        one_class(3 * k + 1, 1)
        one_class(3 * k + 2, 2)
        return carry

    lax.fori_loop(0, _CPW // 3, triple, 0)
    one_class(_CPW - 2, 0)
    one_class(_CPW - 1, 1)

    for b in range(3):
        for j in range(_NSPLIT):
            pltpu.make_async_copy(gbufs[b].at[j], out_u.at[sidxs[b].at[j]],
                                  sem_s[b]).wait()
    pltpu.make_async_copy(prompts_p.at[cls(0)], idx2, sem_i2).wait()

    def ctx_drain(r, carry):
        pltpu.make_async_copy(tmpl, out_u.at[pl.ds(0, 32)], sem_ctx).wait()
        return carry

    lax.fori_loop(0, n_rep, ctx_drain, 0)

    @pl.when(wid < _PFX_WORKERS)
    def _():
        pltpu.make_async_copy(pfx_v, out_u.at[pl.ds(pfx_off, _PFX_CHUNK)],
                              sem_pfx).wait()

    pltpu.async_copy(eos_v, eos.at[pl.ds(base, _CPW)], sem_eos).wait()


@functools.partial(jax.jit, static_argnames=())
def _emb_call(tab_u, prompts_p, pfx_u, ctx_u):
    mesh = plsc.VectorSubcoreMesh(core_axis_name="c", subcore_axis_name="s")
    return pl.kernel(
        _emb_body,
        out_type=[
            jax.ShapeDtypeStruct((_CTX_LEN * _UNITS_PER_T, 128), jnp.float32),
            jax.ShapeDtypeStruct((_NC_PAD,), jnp.int32),
        ],
        mesh=mesh,
        scratch_types=[
            pltpu.VMEM((_NSPLIT, _USPL, 128), jnp.float32),
            pltpu.VMEM((_NSPLIT, _USPL, 128), jnp.float32),
            pltpu.VMEM((_NSPLIT, _USPL, 128), jnp.float32),
            pltpu.VMEM((_NSPLIT, _USPL), jnp.int32),
            pltpu.VMEM((_NSPLIT, _USPL), jnp.int32),
            pltpu.VMEM((_NSPLIT, _USPL), jnp.int32),
            pltpu.VMEM((_NSPLIT, _USPL), jnp.int32),
            pltpu.VMEM((_NSPLIT, _USPL), jnp.int32),
            pltpu.VMEM((_NSPLIT, _USPL), jnp.int32),
            pltpu.VMEM((_NSPLIT, _USPL), jnp.int32),
            pltpu.VMEM((_SUF_PAD,), jnp.int32),
            pltpu.VMEM((_SUF_PAD,), jnp.int32),
            pltpu.VMEM((_SUF_PAD,), jnp.int32),
            pltpu.VMEM((32, 128), jnp.float32),
            pltpu.VMEM((32,), jnp.int32),
            pltpu.VMEM((_PFX_CHUNK, 128), jnp.float32),
            pltpu.VMEM((_CPW,), jnp.int32),
            pltpu.SemaphoreType.DMA,
            pltpu.SemaphoreType.DMA,
            pltpu.SemaphoreType.DMA,
            pltpu.SemaphoreType.DMA,
            pltpu.SemaphoreType.DMA,
            pltpu.SemaphoreType.DMA,
            pltpu.SemaphoreType.DMA,
            pltpu.SemaphoreType.DMA,
            pltpu.SemaphoreType.DMA,
            pltpu.SemaphoreType.DMA,
            pltpu.SemaphoreType.DMA,
            pltpu.SemaphoreType.DMA,
        ],
        compiler_params=pltpu.CompilerParams(use_tc_tiling_on_sc=False,
                                             needs_layout_passes=False),
    )(tab_u, prompts_p, pfx_u, ctx_u)


def kernel(token_embedding, prompts, token_prefix, ctx_embedding):
    tab_u = (token_embedding.reshape(_VOCAB // 8, 8, _JD, 128)
             .transpose(0, 2, 1, 3).reshape(_VOCAB * _JD, 128))
    pfx_u = (token_prefix.reshape(_NC // 8, 8, _JD, 128)
             .transpose(0, 2, 1, 3).reshape(_NC * _JD, 128))
    ctx_u = (ctx_embedding.reshape(_N_CTX // 8, 8, _JD, 128)
             .transpose(0, 2, 1, 3).reshape(_N_CTX * _JD, 128))
    prompts_i = prompts.astype(jnp.int32)
    prompts_p = jnp.pad(prompts_i, ((0, 0), (0, _SUF_PAD - _SUF)),
                        constant_values=-1)
    out_u, eos = _emb_call(tab_u, prompts_p, pfx_u, ctx_u)
    emb = (out_u.reshape(_CTX_LEN, _ROW_TILES, _JD, 8, 128)
           .transpose(1, 3, 0, 2, 4).reshape(_NC, _CTX_LEN, _D))
    return emb, eos[:_NC]


# speedup vs baseline: 1.0728x; 1.0087x over previous
import functools

import jax
import jax.numpy as jnp
from jax import lax
from jax.experimental import pallas as pl
from jax.experimental.pallas import tpu as pltpu
from jax.experimental.pallas import tpu_sc as plsc

_VOCAB = 49408
_D = 512
_NC = 1000
_CTX_LEN = 77
_N_CTX = 16
_SUF = _CTX_LEN - (_N_CTX + 1)
_SUF_PAD = 64
_PREFIX_ROWS = _CTX_LEN - _SUF

_NW = 32
_CPW = 32
_NC_PAD = _NW * _CPW

_JD = _D // 128
_NU_SUF = _SUF * _JD
_NSPLIT = 3
_USPL = _NU_SUF // _NSPLIT
_ROW_TILES = _NC // 8
_UNITS_PER_T = _ROW_TILES * _JD * 8
_PFX_WORKERS = 25
_PFX_CHUNK = _UNITS_PER_T // _PFX_WORKERS


def _emb_body(tab_u, prompts_p, pfx_u, ctx_u, out_u, eos,
              gbuf0, gbuf1, gbuf2, gidx0, gidx1, gidx2,
              sidx0, sidx1, sidx2, sbase,
              idx0, idx1, idx2, tmpl, cidx, pidx, pfx_v, eos_v,
              sem_i0, sem_i1, sem_i2, sem_g0, sem_g1, sem_g2,
              sem_s0, sem_s1, sem_s2,
              sem_ctx, sem_pfx, sem_eos):
    wid = lax.axis_index("s") * 2 + lax.axis_index("c")
    base = wid * _CPW
    gbufs = (gbuf0, gbuf1, gbuf2)
    gidxs = (gidx0, gidx1, gidx2)
    sidxs = (sidx0, sidx1, sidx2)
    idxs = (idx0, idx1, idx2)
    sem_i = (sem_i0, sem_i1, sem_i2)
    sem_g = (sem_g0, sem_g1, sem_g2)
    sem_s = (sem_s0, sem_s1, sem_s2)

    lanes = lax.iota(jnp.int32, 16)

    def cls(i):
        return jnp.minimum(base + i, jnp.int32(_NC - 1))

    pfx_off = jnp.minimum(wid, jnp.int32(_PFX_WORKERS - 1)) * _PFX_CHUNK
    for j in range(2):
        for m in range(5):
            u = pfx_off + 80 * j + 16 * m + lanes
            pidx[j, pl.ds(16 * m, 16)] = (
                32 * (u >> 5) + 4 * (u & 7) + ((u >> 3) & 3))

    @pl.when(wid < _PFX_WORKERS)
    def _():
        for j in range(2):
            pltpu.async_copy(pfx_u.at[pidx.at[j]],
                             pfx_v.at[pl.ds(80 * j, 80)], sem_pfx)

    tct = wid >> 1
    half = wid & 1
    for m in range(2):
        r = lanes + 16 * m
        u = 32 * (tct >> 3) + 8 * (r >> 3) + (tct & 7)
        cidx[pl.ds(16 * m, 16)] = u
    pltpu.async_copy(ctx_u.at[cidx], tmpl, sem_ctx).wait()

    n_rep = jnp.int32(63 - half)
    rep0 = jnp.int32(63) * half
    t_base = jnp.int32(_UNITS_PER_T) * (tct + 1) + 32 * rep0

    def ctx_rep(r, carry):
        off = pl.multiple_of(t_base + 32 * r, 8)
        pltpu.async_copy(tmpl, out_u.at[pl.ds(off, 32)], sem_ctx)
        return carry

    lax.fori_loop(0, n_rep, ctx_rep, 0)

    for j in range(_NSPLIT):
        for m in range(_USPL // 16):
            k = 80 * j + 16 * m + lanes
            sbase[j, pl.ds(16 * m, 16)] = (
                jnp.int32(_UNITS_PER_T) * (_PREFIX_ROWS + (k >> 2))
                + 8 * (k & 3))

    @pl.when(wid < _PFX_WORKERS)
    def _():
        for j in range(2):
            pltpu.make_async_copy(pfx_u.at[pidx.at[j]],
                                  pfx_v.at[pl.ds(80 * j, 80)], sem_pfx).wait()
        pltpu.async_copy(pfx_v, out_u.at[pl.ds(pfx_off, _PFX_CHUNK)], sem_pfx)

    pltpu.async_copy(prompts_p.at[cls(0)], idx0, sem_i0)

    def one_class(i, b):
        c = cls(i)
        offc = 32 * (c >> 3) + (c & 7)
        @pl.when(i >= 3)
        def _():
            for j in range(_NSPLIT):
                pltpu.make_async_copy(gbufs[b].at[j], out_u.at[sidxs[b].at[j]],
                                      sem_s[b]).wait()
        pltpu.make_async_copy(prompts_p.at[c], idxs[b], sem_i[b]).wait()
        for j in range(_NSPLIT):
            for m in range(_USPL // 16):
                k = 80 * j + 16 * m + lanes
                x = plsc.load_gather(idxs[b], [k >> 2])
                gidxs[b][j, pl.ds(16 * m, 16)] = (
                    32 * (x >> 3) + 8 * (k & 3) + (x & 7))
                sidxs[b][j, pl.ds(16 * m, 16)] = (
                    sbase[j, pl.ds(16 * m, 16)] + offc)
            pltpu.async_copy(tab_u.at[gidxs[b].at[j]], gbufs[b].at[j],
                             sem_g[b])
        nb = (b + 1) % 3
        pltpu.async_copy(prompts_p.at[cls(i + 1)], idxs[nb], sem_i[nb])

        best_val = jnp.int32(-2)
        best_pos = jnp.int32(0)
        for m in range(_SUF_PAD // 16):
            vj = idxs[b][pl.ds(16 * m, 16)]
            mj = jnp.max(vj)
            pj = jnp.min(jnp.where(vj == mj, lanes + jnp.int32(16 * m),
                                   jnp.int32(1 << 20)))
            upd = mj > best_val
            best_pos = jnp.where(upd, pj, best_pos)
            best_val = jnp.where(upd, mj, best_val)
        plsc.store_scatter(
            eos_v,
            [jnp.full((16,), i, jnp.int32)],
            jnp.full((16,), best_pos + jnp.int32(_PREFIX_ROWS), jnp.int32),
            mask=lanes == 0,
        )

        for j in range(_NSPLIT):
            pltpu.make_async_copy(tab_u.at[gidxs[b].at[j]], gbufs[b].at[j],
                                  sem_g[b]).wait()
        for j in range(_NSPLIT):
            pltpu.async_copy(gbufs[b].at[j], out_u.at[sidxs[b].at[j]],
                             sem_s[b])

    def triple(k, carry):
        one_class(3 * k, 0)
        one_class(3 * k + 1, 1)
        one_class(3 * k + 2, 2)
        return carry

    lax.fori_loop(0, _CPW // 3, triple, 0)
    one_class(_CPW - 2, 0)
    one_class(_CPW - 1, 1)

    for b in range(3):
        for j in range(_NSPLIT):
            pltpu.make_async_copy(gbufs[b].at[j], out_u.at[sidxs[b].at[j]],
                                  sem_s[b]).wait()
    pltpu.make_async_copy(prompts_p.at[cls(0)], idx2, sem_i2).wait()

    def ctx_drain(r, carry):
        pltpu.make_async_copy(tmpl, out_u.at[pl.ds(0, 32)], sem_ctx).wait()
        return carry

    lax.fori_loop(0, n_rep, ctx_drain, 0)

    @pl.when(wid < _PFX_WORKERS)
    def _():
        pltpu.make_async_copy(pfx_v, out_u.at[pl.ds(pfx_off, _PFX_CHUNK)],
                              sem_pfx).wait()

    pltpu.async_copy(eos_v, eos.at[pl.ds(base, _CPW)], sem_eos).wait()


@functools.partial(jax.jit, static_argnames=())
def _emb_call(tab_u, prompts_p, pfx_u, ctx_u):
    mesh = plsc.VectorSubcoreMesh(core_axis_name="c", subcore_axis_name="s")
    return pl.kernel(
        _emb_body,
        out_type=[
            jax.ShapeDtypeStruct((_CTX_LEN * _UNITS_PER_T, 128), jnp.float32),
            jax.ShapeDtypeStruct((_NC_PAD,), jnp.int32),
        ],
        mesh=mesh,
        scratch_types=[
            pltpu.VMEM((_NSPLIT, _USPL, 128), jnp.float32),
            pltpu.VMEM((_NSPLIT, _USPL, 128), jnp.float32),
            pltpu.VMEM((_NSPLIT, _USPL, 128), jnp.float32),
            pltpu.VMEM((_NSPLIT, _USPL), jnp.int32),
            pltpu.VMEM((_NSPLIT, _USPL), jnp.int32),
            pltpu.VMEM((_NSPLIT, _USPL), jnp.int32),
            pltpu.VMEM((_NSPLIT, _USPL), jnp.int32),
            pltpu.VMEM((_NSPLIT, _USPL), jnp.int32),
            pltpu.VMEM((_NSPLIT, _USPL), jnp.int32),
            pltpu.VMEM((_NSPLIT, _USPL), jnp.int32),
            pltpu.VMEM((_SUF_PAD,), jnp.int32),
            pltpu.VMEM((_SUF_PAD,), jnp.int32),
            pltpu.VMEM((_SUF_PAD,), jnp.int32),
            pltpu.VMEM((32, 128), jnp.float32),
            pltpu.VMEM((32,), jnp.int32),
            pltpu.VMEM((2, 80), jnp.int32),
            pltpu.VMEM((_PFX_CHUNK, 128), jnp.float32),
            pltpu.VMEM((_CPW,), jnp.int32),
            pltpu.SemaphoreType.DMA,
            pltpu.SemaphoreType.DMA,
            pltpu.SemaphoreType.DMA,
            pltpu.SemaphoreType.DMA,
            pltpu.SemaphoreType.DMA,
            pltpu.SemaphoreType.DMA,
            pltpu.SemaphoreType.DMA,
            pltpu.SemaphoreType.DMA,
            pltpu.SemaphoreType.DMA,
            pltpu.SemaphoreType.DMA,
            pltpu.SemaphoreType.DMA,
            pltpu.SemaphoreType.DMA,
        ],
        compiler_params=pltpu.CompilerParams(use_tc_tiling_on_sc=False,
                                             needs_layout_passes=False),
    )(tab_u, prompts_p, pfx_u, ctx_u)


def kernel(token_embedding, prompts, token_prefix, ctx_embedding):
    tab_u = (token_embedding.reshape(_VOCAB // 8, 8, _JD, 128)
             .transpose(0, 2, 1, 3).reshape(_VOCAB * _JD, 128))
    pfx_u = token_prefix.reshape(_NC * _JD, 128)
    ctx_u = (ctx_embedding.reshape(_N_CTX // 8, 8, _JD, 128)
             .transpose(0, 2, 1, 3).reshape(_N_CTX * _JD, 128))
    prompts_i = prompts.astype(jnp.int32)
    prompts_p = jnp.pad(prompts_i, ((0, 0), (0, _SUF_PAD - _SUF)),
                        constant_values=-1)
    out_u, eos = _emb_call(tab_u, prompts_p, pfx_u, ctx_u)
    emb = (out_u.reshape(_CTX_LEN, _ROW_TILES, _JD, 8, 128)
           .transpose(1, 3, 0, 2, 4).reshape(_NC, _CTX_LEN, _D))
    return emb, eos[:_NC]
